# Initial kernel scaffold; baseline (speedup 1.0000x reference)
import jax
import jax.numpy as jnp
from jax.experimental import pallas as pl


def kernel(x, edge_index, W1, b1, W2, b2, W3, b3, Wf1, bf1, Wf2, bf2):
    raise NotImplementedError("write your pallas kernel here")


# speedup vs baseline: 5.7987x; 5.7987x over previous
import functools

import jax
import jax.numpy as jnp
from jax import lax
from jax.experimental import pallas as pl
from jax.experimental.pallas import tpu as pltpu
from jax.experimental.pallas import tpu_sc as plsc

N = 10000
E = 160000
D = 256
H = 128
NC = 2
NS = 16
CH = 128
EP = 163840
SP_CHUNKS = EP // NS // CH
HI_CHUNKS = EP // (NC * NS) // CH
TRASH = N
ACC_ROWS = 10240
ZPT = ACC_ROWS // NS
OPT = N // NS
R = 1000

_mesh = plsc.VectorSubcoreMesh(core_axis_name="c", subcore_axis_name="s")


@functools.partial(
    pl.kernel,
    out_type=jax.ShapeDtypeStruct((NC, ACC_ROWS, H), jnp.float32),
    mesh=_mesh,
    scratch_types=[
        pltpu.VMEM((HI_CHUNKS, CH), jnp.int32),
        pltpu.VMEM((CH, H), jnp.float32),
        pltpu.VMEM((16, H), jnp.float32),
        pltpu.VMEM_SHARED((ACC_ROWS, H), jnp.float32),
    ],
)
def _hist(dst_hbm, out_hbm, dst_v, ones_v, zer_v, acc_sh):
    c = lax.axis_index("c")
    s = lax.axis_index("s")
    pltpu.sync_copy(dst_hbm.at[c, s], dst_v)
    one16 = jnp.full((16,), 1.0, jnp.float32)
    z16 = jnp.zeros((16,), jnp.float32)
    for i in range(16):
        for j in range(H // 16):
            zer_v[i, pl.ds(j * 16, 16)] = z16

    def oinit(i, carry):
        for j in range(H // 16):
            ones_v[i, pl.ds(j * 16, 16)] = one16
        return carry
    lax.fori_loop(0, CH, oinit, 0)

    def zcopy(i, carry):
        pltpu.sync_copy(zer_v, acc_sh.at[pl.ds(s * ZPT + i * 16, 16)])
        return carry
    lax.fori_loop(0, ZPT // 16, zcopy, 0)
    plsc.subcore_barrier()

    def body(j, carry):
        pltpu.sync_copy(ones_v, acc_sh.at[dst_v.at[j]], add=True)
        return carry
    lax.fori_loop(0, HI_CHUNKS, body, 0)
    plsc.subcore_barrier()
    pltpu.sync_copy(acc_sh.at[pl.ds(s * ZPT, ZPT)], out_hbm.at[c, pl.ds(s * ZPT, ZPT)])


@functools.partial(
    pl.kernel,
    out_type=jax.ShapeDtypeStruct((NC, N, H), jnp.float32),
    mesh=_mesh,
    scratch_types=[
        pltpu.VMEM((SP_CHUNKS, CH), jnp.int32),
        pltpu.VMEM((SP_CHUNKS, CH), jnp.int32),
        pltpu.VMEM((CH, H), jnp.float32),
        pltpu.VMEM((16, H), jnp.float32),
        pltpu.VMEM_SHARED((ACC_ROWS, H), jnp.float32),
        pltpu.SemaphoreType.DMA,
    ],
)
def _spmm(src_hbm, dst_hbm, g_hbm, out_hbm, src_v, dst_v, rows_v, zer_v, acc_sh, sem):
    c = lax.axis_index("c")
    s = lax.axis_index("s")
    pltpu.sync_copy(src_hbm.at[c, s], src_v)
    pltpu.sync_copy(dst_hbm.at[s], dst_v)
    z16 = jnp.zeros((16,), jnp.float32)

    def zinit(i, carry):
        for j in range(H // 16):
            zer_v[i, pl.ds(j * 16, 16)] = z16
        return carry
    lax.fori_loop(0, 16, zinit, 0)

    def zcopy(i, carry):
        pltpu.sync_copy(zer_v, acc_sh.at[pl.ds(s * ZPT + i * 16, 16)])
        return carry
    lax.fori_loop(0, ZPT // 16, zcopy, 0)
    plsc.subcore_barrier()

    def body(j, carry):
        pltpu.async_copy(g_hbm.at[src_v.at[j]], rows_v, sem).wait()
        pltpu.sync_copy(rows_v, acc_sh.at[dst_v.at[j]], add=True)
        return carry
    lax.fori_loop(0, SP_CHUNKS, body, 0)
    plsc.subcore_barrier()

    @pl.when(s < NS - 1)
    def _():
        pltpu.sync_copy(acc_sh.at[pl.ds(s * ZPT, ZPT)], out_hbm.at[c, pl.ds(s * ZPT, ZPT)])

    @pl.when(s == NS - 1)
    def _():
        pltpu.sync_copy(acc_sh.at[pl.ds((NS - 1) * ZPT, N - (NS - 1) * ZPT)],
                        out_hbm.at[c, pl.ds((NS - 1) * ZPT, N - (NS - 1) * ZPT)])


def _dinv_of(deg_ref):
    return lax.rsqrt(1.0 + deg_ref[:, 0:1] + deg_ref[:, 1:2])


def _mm1_body(x_ref, w_ref, deg_ref, g_ref):
    dinv = _dinv_of(deg_ref)
    h = jnp.dot(x_ref[...], w_ref[...], preferred_element_type=jnp.float32)
    g = h * dinv
    g_ref[0, :, :] = g[:, :H]
    g_ref[1, :, :] = g[:, H:]


def _mid_body(s_ref, g_ref, deg_ref, b_ref, w_ref, o_ref):
    dinv = _dinv_of(deg_ref)
    h = jnp.concatenate([s_ref[0] + g_ref[0], s_ref[1] + g_ref[1]], axis=-1)
    h = jnp.maximum(h * dinv + b_ref[...], 0.0)
    g = jnp.dot(h, w_ref[...], preferred_element_type=jnp.float32) * dinv
    o_ref[0, :, :] = g[:, :H]
    o_ref[1, :, :] = g[:, H:]


def _fin_body(s_ref, g_ref, deg_ref, b3_ref, wf1_ref, bf1_ref, wf2_ref, bf2_ref, o_ref):
    dinv = _dinv_of(deg_ref)
    h = jnp.concatenate([s_ref[0] + g_ref[0], s_ref[1] + g_ref[1]], axis=-1)
    h = jnp.maximum(h * dinv + b3_ref[...], 0.0)
    t = jnp.dot(h, wf1_ref[...], preferred_element_type=jnp.float32)
    t = jnp.maximum(t + bf1_ref[...], 0.0)
    o_ref[...] = jnp.dot(t, wf2_ref[...], preferred_element_type=jnp.float32) + bf2_ref[...]


_grid = (N // R,)
_spec_rows2 = pl.BlockSpec((R, 2), lambda i: (i, 0))
_spec_half = pl.BlockSpec((NC, R, H), lambda i: (0, i, 0))
_spec_full = lambda shape: pl.BlockSpec(shape, lambda i: tuple(0 for _ in shape))

_mm1 = pl.pallas_call(
    _mm1_body,
    grid=_grid,
    in_specs=[
        pl.BlockSpec((R, D), lambda i: (i, 0)),
        _spec_full((D, D)),
        _spec_rows2,
    ],
    out_specs=_spec_half,
    out_shape=jax.ShapeDtypeStruct((NC, N, H), jnp.float32),
)

_mid = pl.pallas_call(
    _mid_body,
    grid=_grid,
    in_specs=[
        _spec_half,
        _spec_half,
        _spec_rows2,
        _spec_full((1, D)),
        _spec_full((D, D)),
    ],
    out_specs=_spec_half,
    out_shape=jax.ShapeDtypeStruct((NC, N, H), jnp.float32),
)

_fin = pl.pallas_call(
    _fin_body,
    grid=_grid,
    in_specs=[
        _spec_half,
        _spec_half,
        _spec_rows2,
        _spec_full((1, D)),
        _spec_full((D, D)),
        _spec_full((1, D)),
        _spec_full((D, 4)),
        _spec_full((1, 4)),
    ],
    out_specs=pl.BlockSpec((R, 4), lambda i: (i, 0)),
    out_shape=jax.ShapeDtypeStruct((N, 4), jnp.float32),
)


def kernel(x, edge_index, W1, b1, W2, b2, W3, b3, Wf1, bf1, Wf2, bf2):
    src = edge_index[0]
    dst = edge_index[1]
    pad = EP - E
    srcp = jnp.concatenate([src, jnp.zeros((pad,), jnp.int32)])
    dstp = jnp.concatenate([dst, jnp.full((pad,), TRASH, jnp.int32)])
    dst_h = dstp.reshape(NC, NS, HI_CHUNKS, CH)
    src2 = jnp.stack([srcp, srcp + N]).reshape(NC, NS, SP_CHUNKS, CH)
    dst_t = dstp.reshape(NS, SP_CHUNKS, CH)

    degp = _hist(dst_h)
    degT = jnp.transpose(degp[:, :N, 0])

    g1 = _mm1(x, W1, degT)
    s1 = _spmm(src2, dst_t, g1.reshape(NC * N, H))
    g2 = _mid(s1, g1, degT, b1.reshape(1, D), W2)
    s2 = _spmm(src2, dst_t, g2.reshape(NC * N, H))
    g3 = _mid(s2, g2, degT, b2.reshape(1, D), W3)
    s3 = _spmm(src2, dst_t, g3.reshape(NC * N, H))
    pred = _fin(s3, g3, degT, b3.reshape(1, D), Wf1, bf1.reshape(1, D),
                Wf2, bf2.reshape(1, 4))
    return pred
